# Initial kernel scaffold; baseline (speedup 1.0000x reference)
import jax
import jax.numpy as jnp
from jax.experimental import pallas as pl


def kernel(edge_index, input, state, W_in, W_rec, leakage):
    raise NotImplementedError("write your pallas kernel here")


# speedup vs baseline: 5.5398x; 5.5398x over previous
import functools

import jax
import jax.numpy as jnp
from jax import lax
from jax.experimental import pallas as pl
from jax.experimental.pallas import tpu as pltpu
from jax.experimental.pallas import tpu_sc as plsc

N_NODES = 10000
N_EDGES = 320000
FEAT = 128
NUM_CORES = 2
NUM_SUBCORES = 16
NUM_TILES = NUM_CORES * NUM_SUBCORES
EDGES_PER_TILE = N_EDGES // NUM_TILES
CHUNK = 80
CHUNKS_PER_TILE = EDGES_PER_TILE // CHUNK
N_PAD = 10240
ROWS_PER_TILE = N_PAD // NUM_SUBCORES
ZROWS = 128


def _sc_body(src_hbm, dst_hbm, state_hbm, out0, out1,
             idx_s, idx_d, rows, zbuf, sem, shared):
    cid = lax.axis_index("c")
    sid = lax.axis_index("s")
    wid = cid * NUM_SUBCORES + sid

    zeros16 = jnp.zeros((16,), jnp.float32)

    def _zrow(r, _):
        def _zcol(j, _):
            zbuf[r, pl.ds(j * 16, 16)] = zeros16
            return 0
        return lax.fori_loop(0, FEAT // 16, _zcol, 0)

    lax.fori_loop(0, ZROWS, _zrow, 0)

    row0 = sid * ROWS_PER_TILE
    for b in range(ROWS_PER_TILE // ZROWS):
        pltpu.sync_copy(zbuf, shared.at[pl.ds(row0 + b * ZROWS, ZROWS)])
    plsc.subcore_barrier()

    ebase = wid * EDGES_PER_TILE

    def _edge_chunk(c, _):
        off = ebase + c * CHUNK
        pltpu.sync_copy(src_hbm.at[pl.ds(off, CHUNK)], idx_s)
        pltpu.sync_copy(dst_hbm.at[pl.ds(off, CHUNK)], idx_d)
        pltpu.async_copy(state_hbm.at[idx_s], rows, sem).wait()
        pltpu.sync_copy(rows, shared.at[idx_d], add=True)
        return 0

    lax.fori_loop(0, CHUNKS_PER_TILE, _edge_chunk, 0)
    plsc.subcore_barrier()

    @pl.when(cid == 0)
    def _():
        pltpu.sync_copy(shared.at[pl.ds(row0, ROWS_PER_TILE)],
                        out0.at[pl.ds(row0, ROWS_PER_TILE)])

    @pl.when(cid == 1)
    def _():
        pltpu.sync_copy(shared.at[pl.ds(row0, ROWS_PER_TILE)],
                        out1.at[pl.ds(row0, ROWS_PER_TILE)])


@jax.jit
def _sc_scatter(src, dst, state):
    mesh = plsc.VectorSubcoreMesh(core_axis_name="c", subcore_axis_name="s")
    f = pl.kernel(
        _sc_body,
        out_type=[jax.ShapeDtypeStruct((N_PAD, FEAT), jnp.float32),
                  jax.ShapeDtypeStruct((N_PAD, FEAT), jnp.float32)],
        mesh=mesh,
        scratch_types=[
            pltpu.VMEM((CHUNK,), jnp.int32),
            pltpu.VMEM((CHUNK,), jnp.int32),
            pltpu.VMEM((CHUNK, FEAT), jnp.float32),
            pltpu.VMEM((ZROWS, FEAT), jnp.float32),
            pltpu.SemaphoreType.DMA,
            pltpu.VMEM_SHARED((N_PAD, FEAT), jnp.float32),
        ],
    )
    return f(src, dst, state)


def _tc_body(leak_ref, x_ref, s_ref, p0_ref, p1_ref, win_ref, wrec_ref, o_ref):
    aggr = p0_ref[...] + p1_ref[...]
    dn = (((1,), (1,)), ((), ()))
    pre = lax.dot_general(x_ref[...], win_ref[...], dn,
                          preferred_element_type=jnp.float32)
    pre = pre + lax.dot_general(aggr, wrec_ref[...], dn,
                                preferred_element_type=jnp.float32)
    lam = leak_ref[0, 0]
    o_ref[...] = lam * jnp.tanh(pre) + (1.0 - lam) * s_ref[...]


@jax.jit
def _tc_dense(leak, x, s, p0, p1, W_in, W_rec):
    blk = 1000
    grid = (N_NODES // blk,)
    row_spec = pl.BlockSpec((blk, FEAT), lambda i: (i, 0))
    w_spec = pl.BlockSpec((FEAT, FEAT), lambda i: (0, 0))
    return pl.pallas_call(
        _tc_body,
        grid=grid,
        in_specs=[
            pl.BlockSpec(memory_space=pltpu.SMEM),
            row_spec, row_spec, row_spec, row_spec, w_spec, w_spec,
        ],
        out_specs=row_spec,
        out_shape=jax.ShapeDtypeStruct((N_NODES, FEAT), jnp.float32),
    )(leak, x, s, p0, p1, W_in, W_rec)


def kernel(edge_index, input, state, W_in, W_rec, leakage):
    src = edge_index[0].astype(jnp.int32)
    dst = edge_index[1].astype(jnp.int32)
    p0, p1 = _sc_scatter(src, dst, state)
    leak2d = jnp.asarray(leakage, jnp.float32).reshape(1, 1)
    return _tc_dense(leak2d, input, state, p0, p1, W_in, W_rec)
